# baseline (speedup 1.0000x reference)
import functools
import jax
import jax.numpy as jnp
from jax import lax
from jax.experimental import pallas as pl
from jax.experimental.pallas import tpu as pltpu
from jax.experimental.pallas import tpu_sc as plsc

N = 8192
P, D, T = 1024, 512, 512
H = 512
E = 16
TN = 1024

NC, NS = 2, 16
NW = NC * NS
ROWS = N // NW
GRP = ROWS // 16


def _mlp_kernel(xp_ref, xd_ref, xt_ref, w1_ref, b1_ref, a_ref, g_ref,
                bb_ref, w2_ref, b2_ref, out_ref):
    dn = (((1,), (1,)), ((), ()))
    x = jnp.concatenate([xp_ref[...], xd_ref[...], xt_ref[...]], axis=1)
    h = jax.lax.dot_general(x, w1_ref[...], dn,
                            preferred_element_type=jnp.float32)
    h = h + b1_ref[...][None, :]
    a = a_ref[0]
    h = jnp.maximum(h, 0.0) + a * jnp.minimum(h, 0.0)
    mu = jnp.mean(h, axis=-1, keepdims=True)
    c = h - mu
    var = jnp.mean(c * c, axis=-1, keepdims=True)
    hn = c / jnp.sqrt(var + 1e-5) * g_ref[...][None, :] + bb_ref[...][None, :]
    logits = jax.lax.dot_general(hn, w2_ref[...], dn,
                                 preferred_element_type=jnp.float32)
    logits = logits + b2_ref[...][None, :]
    out_ref[...] = logits.T


def _mlp_logits_t(protein_raw, v_prior, trust_vector, W1, b1, prelu_a, ln_g,
                  ln_b, W2, b2):
    grid = (N // TN,)
    full = lambda i: (0, 0)
    row = lambda i: (i, 0)
    vec = lambda i: (0,)
    return pl.pallas_call(
        _mlp_kernel,
        grid=grid,
        in_specs=[
            pl.BlockSpec((TN, P), row),
            pl.BlockSpec((TN, D), row),
            pl.BlockSpec((TN, T), row),
            pl.BlockSpec((H, P + D + T), full),
            pl.BlockSpec((H,), vec),
            pl.BlockSpec(memory_space=pltpu.SMEM),
            pl.BlockSpec((H,), vec),
            pl.BlockSpec((H,), vec),
            pl.BlockSpec((E, H), full),
            pl.BlockSpec((E,), vec),
        ],
        out_specs=pl.BlockSpec((E, TN), lambda i: (0, i)),
        out_shape=jax.ShapeDtypeStruct((E, N), jnp.float32),
        compiler_params=pltpu.CompilerParams(dimension_semantics=("parallel",)),
    )(protein_raw, v_prior, trust_vector, W1, b1, prelu_a.reshape(1), ln_g,
      ln_b, W2, b2)


@functools.partial(
    pl.kernel,
    mesh=plsc.VectorSubcoreMesh(core_axis_name="c", subcore_axis_name="s"),
    out_type=jax.ShapeDtypeStruct((E, N), jnp.float32),
    scratch_types=[
        pltpu.VMEM((E, ROWS), jnp.float32),
        pltpu.VMEM((E, ROWS), jnp.float32),
    ],
)
def _sc_route(lgt_hbm, out_hbm, lg_v, pr_v):
    wid = lax.axis_index("s") * NC + lax.axis_index("c")
    base = wid * ROWS
    pltpu.sync_copy(lgt_hbm.at[:, pl.ds(base, ROWS)], lg_v)
    ninf = jnp.full((16,), -jnp.inf, jnp.float32)
    zero = jnp.full((16,), 0.0, jnp.float32)
    one = jnp.full((16,), 1.0, jnp.float32)
    sent = jnp.full((16,), E, jnp.int32)
    for j in range(GRP):
        sl = pl.ds(j * 16, 16)
        vs = [lg_v[e, sl] for e in range(E)]
        m1 = vs[0]
        for e in range(1, E):
            m1 = jnp.maximum(m1, vs[e])
        idx1 = sent
        for e in range(E - 1, -1, -1):
            idx1 = jnp.where(vs[e] == m1, jnp.full((16,), e, jnp.int32), idx1)
        ms = [jnp.where(idx1 == jnp.full((16,), e, jnp.int32), ninf, vs[e])
              for e in range(E)]
        m2 = ms[0]
        for e in range(1, E):
            m2 = jnp.maximum(m2, ms[e])
        idx2 = sent
        for e in range(E - 1, -1, -1):
            idx2 = jnp.where(ms[e] == m2, jnp.full((16,), e, jnp.int32), idx2)
        e2 = jnp.exp(m2 - m1)
        z = one + e2
        p1 = one / z
        p2 = e2 / z
        for e in range(E):
            ev = jnp.full((16,), e, jnp.int32)
            pr_v[e, sl] = jnp.where(idx1 == ev, p1,
                                    jnp.where(idx2 == ev, p2, zero))
    pltpu.sync_copy(pr_v, out_hbm.at[:, pl.ds(base, ROWS)])


def kernel(protein_raw, v_prior, trust_vector, W1, b1, prelu_a, ln_g, ln_b,
           W2, b2):
    lgt = _mlp_logits_t(protein_raw, v_prior, trust_vector, W1, b1, prelu_a,
                        ln_g, ln_b, W2, b2)
    return _sc_route(lgt).T


# speedup vs baseline: 1.5303x; 1.5303x over previous
import jax
import jax.numpy as jnp
from jax.experimental import pallas as pl
from jax.experimental.pallas import tpu as pltpu

N = 8192
P, D, T = 1024, 512, 512
H = 512
E = 16
TN = 1024


def _gate_kernel(xp_ref, xd_ref, xt_ref, w1_ref, b1_ref, a_ref, g_ref,
                 bb_ref, w2_ref, b2_ref, out_ref):
    dn = (((1,), (1,)), ((), ()))
    x = jnp.concatenate([xp_ref[...], xd_ref[...], xt_ref[...]], axis=1)
    h = jax.lax.dot_general(x, w1_ref[...], dn,
                            preferred_element_type=jnp.float32)
    h = h + b1_ref[...][None, :]
    a = a_ref[0]
    h = jnp.maximum(h, 0.0) + a * jnp.minimum(h, 0.0)
    mu = jnp.mean(h, axis=-1, keepdims=True)
    c = h - mu
    var = jnp.mean(c * c, axis=-1, keepdims=True)
    hn = c / jnp.sqrt(var + 1e-5) * g_ref[...][None, :] + bb_ref[...][None, :]
    logits = jax.lax.dot_general(hn, w2_ref[...], dn,
                                 preferred_element_type=jnp.float32)
    logits = logits + b2_ref[...][None, :]
    lt = logits.T
    iota = jax.lax.broadcasted_iota(jnp.int32, lt.shape, 0)
    m1 = jnp.max(lt, axis=0, keepdims=True)
    idx1 = jnp.min(jnp.where(lt == m1, iota, E), axis=0, keepdims=True)
    is1 = iota == idx1
    masked = jnp.where(is1, -jnp.inf, lt)
    m2 = jnp.max(masked, axis=0, keepdims=True)
    idx2 = jnp.min(jnp.where(masked == m2, iota, E), axis=0, keepdims=True)
    e2 = jnp.exp(m2 - m1)
    z = 1.0 + e2
    pt = jnp.where(is1, 1.0 / z, jnp.where(iota == idx2, e2 / z, 0.0))
    out_ref[...] = pt


def kernel(protein_raw, v_prior, trust_vector, W1, b1, prelu_a, ln_g, ln_b,
           W2, b2):
    grid = (N // TN,)
    full = lambda i: (0, 0)
    row = lambda i: (i, 0)
    vec = lambda i: (0,)
    return pl.pallas_call(
        _gate_kernel,
        grid=grid,
        in_specs=[
            pl.BlockSpec((TN, P), row),
            pl.BlockSpec((TN, D), row),
            pl.BlockSpec((TN, T), row),
            pl.BlockSpec((H, P + D + T), full),
            pl.BlockSpec((H,), vec),
            pl.BlockSpec(memory_space=pltpu.SMEM),
            pl.BlockSpec((H,), vec),
            pl.BlockSpec((H,), vec),
            pl.BlockSpec((E, H), full),
            pl.BlockSpec((E,), vec),
        ],
        out_specs=pl.BlockSpec((E, TN), lambda i: (0, i)),
        out_shape=jax.ShapeDtypeStruct((E, N), jnp.float32),
        compiler_params=pltpu.CompilerParams(dimension_semantics=("parallel",)),
    )(protein_raw, v_prior, trust_vector, W1, b1, prelu_a.reshape(1), ln_g,
      ln_b, W2, b2).T
